# baseline (speedup 1.0000x reference)
import functools

import jax
import jax.numpy as jnp
from jax import lax
from jax.experimental import pallas as pl
from jax.experimental.pallas import tpu as pltpu
from jax.experimental.pallas import tpu_sc as plsc

B = 2
S = 2048
T = B * S
HIDDEN = 1024
FF = 4096
FF_S = FF // 2
E = 8
K = 2
AUX_COEF = 0.01
Z_COEF = 0.001

P = T * K
BLK = 512
NBUF = P + E * BLK
NBLOCKS = NBUF // BLK
NW = 32

_INTERPRET = False


_RT_BLK = 1024


def _router_body(x_ref, wr_ref, e0_ref, e1_ref, w0_ref, w1_ref,
                 aux_ref, z_ref, ent_ref, acc_ref, sacc_ref):
    i = pl.program_id(0)
    nb = pl.num_programs(0)

    @pl.when(i == 0)
    def _():
        acc_ref[...] = jnp.zeros_like(acc_ref)
        sacc_ref[0] = 0.0
        sacc_ref[1] = 0.0

    x = x_ref[...]
    logits = jnp.dot(x, wr_ref[...], preferred_element_type=jnp.float32)
    m = jnp.max(logits, axis=-1, keepdims=True)
    ex = jnp.exp(logits - m)
    s = jnp.sum(ex, axis=-1, keepdims=True)
    probs = ex / s

    lanes = jax.lax.broadcasted_iota(jnp.int32, probs.shape, 1)
    i1 = jnp.argmax(probs, axis=-1)[:, None].astype(jnp.int32)
    v1 = jnp.max(probs, axis=-1, keepdims=True)
    masked = jnp.where(lanes == i1, -jnp.inf, probs)
    i2 = jnp.argmax(masked, axis=-1)[:, None].astype(jnp.int32)
    v2 = jnp.max(masked, axis=-1, keepdims=True)
    tot = v1 + v2
    e0_ref[...] = i1
    e1_ref[...] = i2
    w0_ref[...] = v1 / tot
    w1_ref[...] = v2 / tot

    oh1 = (lanes == i1).astype(jnp.float32)
    oh2 = (lanes == i2).astype(jnp.float32)
    z = jnp.log(s[:, 0]) + m[:, 0]
    ent = -jnp.sum(probs * jnp.log(probs + 1e-9), axis=-1)
    acc_ref[0, :] += jnp.sum(oh1 + oh2, axis=0)
    acc_ref[1, :] += jnp.sum(probs, axis=0)
    sacc_ref[0] += jnp.sum(z * z)
    sacc_ref[1] += jnp.sum(ent)

    @pl.when(i == nb - 1)
    def _():
        frac = acc_ref[0, :] / T
        mean_prob = acc_ref[1, :] / T
        aux_ref[0, 0] = AUX_COEF * E * jnp.sum(frac * mean_prob)
        z_ref[0, 0] = Z_COEF * sacc_ref[0] / T
        ent_ref[0, 0] = sacc_ref[1] / T


def _run_router(flat, w_router):
    nb = T // _RT_BLK
    return pl.pallas_call(
        _router_body,
        grid=(nb,),
        in_specs=[
            pl.BlockSpec((_RT_BLK, HIDDEN), lambda i: (i, 0)),
            pl.BlockSpec((HIDDEN, E), lambda i: (0, 0)),
        ],
        out_specs=[
            pl.BlockSpec((_RT_BLK, 1), lambda i: (i, 0)),
            pl.BlockSpec((_RT_BLK, 1), lambda i: (i, 0)),
            pl.BlockSpec((_RT_BLK, 1), lambda i: (i, 0)),
            pl.BlockSpec((_RT_BLK, 1), lambda i: (i, 0)),
            pl.BlockSpec(memory_space=pltpu.SMEM),
            pl.BlockSpec(memory_space=pltpu.SMEM),
            pl.BlockSpec(memory_space=pltpu.SMEM),
        ],
        out_shape=[
            jax.ShapeDtypeStruct((T, 1), jnp.int32),
            jax.ShapeDtypeStruct((T, 1), jnp.int32),
            jax.ShapeDtypeStruct((T, 1), jnp.float32),
            jax.ShapeDtypeStruct((T, 1), jnp.float32),
            jax.ShapeDtypeStruct((1, 1), jnp.float32),
            jax.ShapeDtypeStruct((1, 1), jnp.float32),
            jax.ShapeDtypeStruct((1, 1), jnp.float32),
        ],
        scratch_shapes=[pltpu.VMEM((2, E), jnp.float32),
                        pltpu.SMEM((2,), jnp.float32)],
        interpret=_INTERPRET,
    )(flat, w_router)


_PB = 512


def _meta_body(e0_ref, e1_ref, pos_ref, blke_ref, m_ref):
    ef = jnp.concatenate([e0_ref[...], e1_ref[...]], axis=0)
    lanes = jax.lax.broadcasted_iota(jnp.int32, (P, E), 1)
    m_ref[...] = (ef == lanes).astype(jnp.float32)

    counts = jnp.sum(m_ref[...], axis=0)[None, :]
    nb = jnp.floor((counts + (BLK - 1)) * (1.0 / BLK))
    uidx = jax.lax.broadcasted_iota(jnp.int32, (E, E), 0)
    ujdx = jax.lax.broadcasted_iota(jnp.int32, (E, E), 1)
    upper = (uidx < ujdx).astype(jnp.float32)
    offs = BLK * jnp.dot(nb, upper, preferred_element_type=jnp.float32)

    ri = jax.lax.broadcasted_iota(jnp.int32, (_PB, _PB), 0)
    rj = jax.lax.broadcasted_iota(jnp.int32, (_PB, _PB), 1)
    ltri = (rj < ri).astype(jnp.float32)

    def blk_body(b, run):
        mb = m_ref[pl.ds(b * _PB, _PB), :]
        pre = jnp.dot(ltri, mb, preferred_element_type=jnp.float32) + run
        posb = jnp.sum(mb * (pre + offs), axis=1, keepdims=True)
        pos_ref[pl.ds(b * _PB, _PB), :] = posb.astype(jnp.int32)
        return run + jnp.sum(mb, axis=0, keepdims=True)

    lax.fori_loop(0, P // _PB, blk_body, jnp.zeros((1, E), jnp.float32))

    bi = jax.lax.broadcasted_iota(jnp.int32, (NBLOCKS, E), 0).astype(jnp.float32)
    starts = offs * (1.0 / BLK)
    blke = jnp.sum((bi >= starts).astype(jnp.int32), axis=1, keepdims=True) - 1
    blke_ref[...] = blke


def _run_meta(e0, e1):
    return pl.pallas_call(
        _meta_body,
        out_shape=[
            jax.ShapeDtypeStruct((P, 1), jnp.int32),
            jax.ShapeDtypeStruct((NBLOCKS, 1), jnp.int32),
        ],
        scratch_shapes=[pltpu.VMEM((P, E), jnp.float32)],
        interpret=_INTERPRET,
    )(e0, e1)


_SC_MESH = dict(core_axis_name="c", subcore_axis_name="s")
_SPT = NBUF // NW
_GCH = 48
_NCH = _SPT // _GCH


def _sc_dispatch_body(pos_hbm, w_hbm, flat_hbm, xs_hbm, ws_hbm,
                      pos_c, w_c, inv_v, ws_v, rows0, rows1,
                      gsem0, gsem1, wsem0, wsem1):
    wid = lax.axis_index("s") * 2 + lax.axis_index("c")
    base = wid * _SPT

    zf = jnp.zeros((16,), jnp.float32)

    def init_body(i, c):
        spread = (jax.lax.iota(jnp.int32, 16) * 64 + base + i * 16) & (T - 1)
        inv_v[pl.ds(i * 16, 16)] = spread
        ws_v[pl.ds(i * 16, 16)] = zf
        return c

    lax.fori_loop(0, _SPT // 16, init_body, 0)

    def chunk_body(cc, c):
        pltpu.sync_copy(pos_hbm.at[pl.ds(cc * 512, 512)], pos_c)
        pltpu.sync_copy(w_hbm.at[pl.ds(cc * 512, 512)], w_c)

        def sc_body(j, c2):
            idx = pos_c[pl.ds(j * 16, 16)] - base
            wv = w_c[pl.ds(j * 16, 16)]
            msk = (idx >= 0) & (idx < _SPT)
            tok = (jax.lax.iota(jnp.int32, 16) + (cc * 512 + j * 16)) & (T - 1)
            plsc.store_scatter(inv_v, [idx], tok, mask=msk)
            plsc.store_scatter(ws_v, [idx], wv, mask=msk)
            return c2

        lax.fori_loop(0, 512 // 16, sc_body, 0)
        return c

    lax.fori_loop(0, P // 512, chunk_body, 0)

    bufs = (rows0, rows1)
    gsems = (gsem0, gsem1)
    wsems = (wsem0, wsem1)
    gcp = [None, None]
    wcp = [None, None]
    for c in range(_NCH):
        b = c % 2
        if wcp[b] is not None:
            wcp[b].wait()
        gcp[b] = pltpu.async_copy(
            flat_hbm.at[inv_v.at[pl.ds(c * _GCH, _GCH)]], bufs[b], gsems[b])
        if c > 0:
            pb = (c - 1) % 2
            gcp[pb].wait()
            wcp[pb] = pltpu.async_copy(
                bufs[pb], xs_hbm.at[pl.ds(base + (c - 1) * _GCH, _GCH)],
                wsems[pb])
    lb = (_NCH - 1) % 2
    gcp[lb].wait()
    pltpu.sync_copy(bufs[lb], xs_hbm.at[pl.ds(base + (_NCH - 1) * _GCH, _GCH)])
    if wcp[1 - lb] is not None:
        wcp[1 - lb].wait()
    pltpu.sync_copy(ws_v, ws_hbm.at[pl.ds(base, _SPT)])


def _sc_dispatch(posf, wf, flat):
    k = pl.kernel(
        _sc_dispatch_body,
        out_type=[
            jax.ShapeDtypeStruct((NBUF, HIDDEN), jnp.float32),
            jax.ShapeDtypeStruct((NBUF,), jnp.float32),
        ],
        mesh=plsc.VectorSubcoreMesh(**_SC_MESH),
        compiler_params=pltpu.CompilerParams(needs_layout_passes=False),
        scratch_types=[
            pltpu.VMEM((512,), jnp.int32),
            pltpu.VMEM((512,), jnp.float32),
            pltpu.VMEM((_SPT,), jnp.int32),
            pltpu.VMEM((_SPT,), jnp.float32),
            pltpu.VMEM((_GCH, HIDDEN), jnp.float32),
            pltpu.VMEM((_GCH, HIDDEN), jnp.float32),
            pltpu.SemaphoreType.DMA,
            pltpu.SemaphoreType.DMA,
            pltpu.SemaphoreType.DMA,
            pltpu.SemaphoreType.DMA,
        ],
    )
    return k(posf, wf, flat)


_FB = 1024


def _stage1_body(blke_ref, x_ref, wg_ref, wu_ref, wcol_ref, h_ref):
    x = x_ref[...].astype(jnp.bfloat16)
    g = jnp.dot(x, wg_ref[0].astype(jnp.bfloat16),
                preferred_element_type=jnp.float32)
    u = jnp.dot(x, wu_ref[0].astype(jnp.bfloat16),
                preferred_element_type=jnp.float32)
    h = (g * jax.nn.sigmoid(g)) * u * wcol_ref[...]
    h_ref[...] = h.astype(jnp.bfloat16)


def _run_stage1(blke, xs, wg, wu, wcol):
    grid = (FF // _FB, NBLOCKS)
    return pl.pallas_call(
        _stage1_body,
        grid_spec=pltpu.PrefetchScalarGridSpec(
            num_scalar_prefetch=1,
            grid=grid,
            in_specs=[
                pl.BlockSpec((BLK, HIDDEN), lambda f, b, s: (b, 0)),
                pl.BlockSpec((1, HIDDEN, _FB), lambda f, b, s: (s[b], 0, f)),
                pl.BlockSpec((1, HIDDEN, _FB), lambda f, b, s: (s[b], 0, f)),
                pl.BlockSpec((BLK, 1), lambda f, b, s: (b, 0)),
            ],
            out_specs=pl.BlockSpec((BLK, _FB), lambda f, b, s: (b, f)),
        ),
        out_shape=jax.ShapeDtypeStruct((NBUF, FF), jnp.bfloat16),
        compiler_params=pltpu.CompilerParams(
            dimension_semantics=("arbitrary", "arbitrary"),
        ),
        interpret=_INTERPRET,
    )(blke, xs, wg, wu, wcol)


def _stage2_body(blke_ref, h_ref, wd_ref, out_ref, acc_ref):
    f = pl.program_id(1)

    @pl.when(f == 0)
    def _():
        acc_ref[...] = jnp.zeros_like(acc_ref)

    acc_ref[...] += jnp.dot(h_ref[...], wd_ref[0].astype(jnp.bfloat16),
                            preferred_element_type=jnp.float32)

    @pl.when(f == pl.num_programs(1) - 1)
    def _():
        out_ref[...] = acc_ref[...]


def _run_stage2(blke, h, wd):
    grid = (NBLOCKS, FF // _FB)
    return pl.pallas_call(
        _stage2_body,
        grid_spec=pltpu.PrefetchScalarGridSpec(
            num_scalar_prefetch=1,
            grid=grid,
            in_specs=[
                pl.BlockSpec((BLK, _FB), lambda b, f, s: (b, f)),
                pl.BlockSpec((1, _FB, HIDDEN), lambda b, f, s: (s[b], f, 0)),
            ],
            out_specs=pl.BlockSpec((BLK, HIDDEN), lambda b, f, s: (b, 0)),
            scratch_shapes=[pltpu.VMEM((BLK, HIDDEN), jnp.float32)],
        ),
        out_shape=jax.ShapeDtypeStruct((NBUF, HIDDEN), jnp.float32),
        compiler_params=pltpu.CompilerParams(
            dimension_semantics=("arbitrary", "arbitrary"),
        ),
        interpret=_INTERPRET,
    )(blke, h, wd)


_TBS = 512


def _shared_body(x_ref, wg_ref, wu_ref, wd_ref, out_ref, acc_ref):
    f = pl.program_id(1)

    @pl.when(f == 0)
    def _():
        acc_ref[...] = jnp.zeros_like(acc_ref)

    x = x_ref[...].astype(jnp.bfloat16)
    g = jnp.dot(x, wg_ref[...].astype(jnp.bfloat16),
                preferred_element_type=jnp.float32)
    u = jnp.dot(x, wu_ref[...].astype(jnp.bfloat16),
                preferred_element_type=jnp.float32)
    h = ((g * jax.nn.sigmoid(g)) * u).astype(jnp.bfloat16)
    acc_ref[...] += jnp.dot(h, wd_ref[...].astype(jnp.bfloat16),
                            preferred_element_type=jnp.float32)

    @pl.when(f == pl.num_programs(1) - 1)
    def _():
        out_ref[...] = 0.5 * acc_ref[...]


def _run_shared(flat, wg_s, wu_s, wd_s):
    grid = (T // _TBS, FF_S // _FB)
    return pl.pallas_call(
        _shared_body,
        grid=grid,
        in_specs=[
            pl.BlockSpec((_TBS, HIDDEN), lambda t, f: (t, 0)),
            pl.BlockSpec((HIDDEN, _FB), lambda t, f: (0, f)),
            pl.BlockSpec((HIDDEN, _FB), lambda t, f: (0, f)),
            pl.BlockSpec((_FB, HIDDEN), lambda t, f: (f, 0)),
        ],
        out_specs=pl.BlockSpec((_TBS, HIDDEN), lambda t, f: (t, 0)),
        out_shape=jax.ShapeDtypeStruct((T, HIDDEN), jnp.float32),
        scratch_shapes=[pltpu.VMEM((_TBS, HIDDEN), jnp.float32)],
        compiler_params=pltpu.CompilerParams(
            dimension_semantics=("parallel", "arbitrary"),
        ),
        interpret=_INTERPRET,
    )(flat, wg_s, wu_s, wd_s)


_TPT = T // NW
_CCH = 32


def _sc_combine_body(outs_hbm, p0_hbm, p1_hbm, y0_hbm, y1_hbm,
                     p0_v, p1_v, r0_v, r1_v, sem0, sem1):
    wid = lax.axis_index("s") * 2 + lax.axis_index("c")
    base = wid * _TPT
    pltpu.sync_copy(p0_hbm.at[pl.ds(base, _TPT)], p0_v)
    pltpu.sync_copy(p1_hbm.at[pl.ds(base, _TPT)], p1_v)

    def c_body(c, carry):
        t0 = base + c * _CCH
        cp0 = pltpu.async_copy(outs_hbm.at[p0_v.at[pl.ds(c * _CCH, _CCH)]],
                               r0_v, sem0)
        cp1 = pltpu.async_copy(outs_hbm.at[p1_v.at[pl.ds(c * _CCH, _CCH)]],
                               r1_v, sem1)
        cp0.wait()
        cp1.wait()
        pltpu.sync_copy(r0_v, y0_hbm.at[pl.ds(t0, _CCH)])
        pltpu.sync_copy(r1_v, y1_hbm.at[pl.ds(t0, _CCH)])
        return carry

    lax.fori_loop(0, _TPT // _CCH, c_body, 0)


def _sc_combine(outs, p0, p1):
    k = pl.kernel(
        _sc_combine_body,
        out_type=[
            jax.ShapeDtypeStruct((T, HIDDEN), jnp.float32),
            jax.ShapeDtypeStruct((T, HIDDEN), jnp.float32),
        ],
        mesh=plsc.VectorSubcoreMesh(**_SC_MESH),
        compiler_params=pltpu.CompilerParams(needs_layout_passes=False),
        scratch_types=[
            pltpu.VMEM((_TPT,), jnp.int32),
            pltpu.VMEM((_TPT,), jnp.int32),
            pltpu.VMEM((_CCH, HIDDEN), jnp.float32),
            pltpu.VMEM((_CCH, HIDDEN), jnp.float32),
            pltpu.SemaphoreType.DMA,
            pltpu.SemaphoreType.DMA,
        ],
    )
    return k(outs, p0, p1)


def _final_body(y0_ref, y1_ref, sh_ref, out_ref):
    out_ref[...] = y0_ref[...] + y1_ref[...] + sh_ref[...]


def _run_final(y0, y1, sh):
    grid = (T // _TBS,)
    spec = pl.BlockSpec((_TBS, HIDDEN), lambda t: (t, 0))
    return pl.pallas_call(
        _final_body,
        grid=grid,
        in_specs=[spec, spec, spec],
        out_specs=spec,
        out_shape=jax.ShapeDtypeStruct((T, HIDDEN), jnp.float32),
        compiler_params=pltpu.CompilerParams(
            dimension_semantics=("parallel",),
        ),
        interpret=_INTERPRET,
    )(y0, y1, sh)


def kernel(hidden_states, W_router, Wg, Wu, Wd, Wg_s, Wu_s, Wd_s):
    Bv, Sv, D = hidden_states.shape
    flat = hidden_states.reshape(-1, D)
    e0, e1, w0, w1, aux, z, ent = _run_router(flat, W_router)
    pos2d, blke2d = _run_meta(e0, e1)
    posf = pos2d.reshape(-1)
    blke = blke2d.reshape(-1)
    wf = jnp.concatenate([w0, w1], axis=0).reshape(-1)
    xs, wslot = _sc_dispatch(posf, wf, flat)
    h = _run_stage1(blke, xs, Wg, Wu, wslot.reshape(NBUF, 1))
    outs = _run_stage2(blke, h, Wd)
    sh = _run_shared(flat, Wg_s, Wu_s, Wd_s)
    y0, y1 = _sc_combine(outs, posf[:T], posf[T:])
    out = _run_final(y0, y1, sh)
    return (out.reshape(Bv, Sv, D), aux[0, 0], z[0, 0], ent[0, 0])


# speedup vs baseline: 1.0398x; 1.0398x over previous
import functools

import jax
import jax.numpy as jnp
from jax import lax
from jax.experimental import pallas as pl
from jax.experimental.pallas import tpu as pltpu
from jax.experimental.pallas import tpu_sc as plsc

B = 2
S = 2048
T = B * S
HIDDEN = 1024
FF = 4096
FF_S = FF // 2
E = 8
K = 2
AUX_COEF = 0.01
Z_COEF = 0.001

P = T * K
BLK = 512
NBUF = P + E * BLK
NBLOCKS = NBUF // BLK
NW = 32

_INTERPRET = False


_RT_BLK = 1024


def _router_body(x_ref, wr_ref, e0_ref, e1_ref, w0_ref, w1_ref,
                 aux_ref, z_ref, ent_ref, acc_ref, sacc_ref):
    i = pl.program_id(0)
    nb = pl.num_programs(0)

    @pl.when(i == 0)
    def _():
        acc_ref[...] = jnp.zeros_like(acc_ref)
        sacc_ref[0] = 0.0
        sacc_ref[1] = 0.0

    x = x_ref[...]
    logits = jnp.dot(x, wr_ref[...], preferred_element_type=jnp.float32)
    m = jnp.max(logits, axis=-1, keepdims=True)
    ex = jnp.exp(logits - m)
    s = jnp.sum(ex, axis=-1, keepdims=True)
    probs = ex / s

    lanes = jax.lax.broadcasted_iota(jnp.int32, probs.shape, 1)
    i1 = jnp.argmax(probs, axis=-1)[:, None].astype(jnp.int32)
    v1 = jnp.max(probs, axis=-1, keepdims=True)
    masked = jnp.where(lanes == i1, -jnp.inf, probs)
    i2 = jnp.argmax(masked, axis=-1)[:, None].astype(jnp.int32)
    v2 = jnp.max(masked, axis=-1, keepdims=True)
    tot = v1 + v2
    e0_ref[...] = i1
    e1_ref[...] = i2
    w0_ref[...] = v1 / tot
    w1_ref[...] = v2 / tot

    oh1 = (lanes == i1).astype(jnp.float32)
    oh2 = (lanes == i2).astype(jnp.float32)
    z = jnp.log(s[:, 0]) + m[:, 0]
    ent = -jnp.sum(probs * jnp.log(probs + 1e-9), axis=-1)
    acc_ref[0, :] += jnp.sum(oh1 + oh2, axis=0)
    acc_ref[1, :] += jnp.sum(probs, axis=0)
    sacc_ref[0] += jnp.sum(z * z)
    sacc_ref[1] += jnp.sum(ent)

    @pl.when(i == nb - 1)
    def _():
        frac = acc_ref[0, :] / T
        mean_prob = acc_ref[1, :] / T
        aux_ref[0, 0] = AUX_COEF * E * jnp.sum(frac * mean_prob)
        z_ref[0, 0] = Z_COEF * sacc_ref[0] / T
        ent_ref[0, 0] = sacc_ref[1] / T


def _run_router(flat, w_router):
    nb = T // _RT_BLK
    return pl.pallas_call(
        _router_body,
        grid=(nb,),
        in_specs=[
            pl.BlockSpec((_RT_BLK, HIDDEN), lambda i: (i, 0)),
            pl.BlockSpec((HIDDEN, E), lambda i: (0, 0)),
        ],
        out_specs=[
            pl.BlockSpec((_RT_BLK, 1), lambda i: (i, 0)),
            pl.BlockSpec((_RT_BLK, 1), lambda i: (i, 0)),
            pl.BlockSpec((_RT_BLK, 1), lambda i: (i, 0)),
            pl.BlockSpec((_RT_BLK, 1), lambda i: (i, 0)),
            pl.BlockSpec(memory_space=pltpu.SMEM),
            pl.BlockSpec(memory_space=pltpu.SMEM),
            pl.BlockSpec(memory_space=pltpu.SMEM),
        ],
        out_shape=[
            jax.ShapeDtypeStruct((T, 1), jnp.int32),
            jax.ShapeDtypeStruct((T, 1), jnp.int32),
            jax.ShapeDtypeStruct((T, 1), jnp.float32),
            jax.ShapeDtypeStruct((T, 1), jnp.float32),
            jax.ShapeDtypeStruct((1, 1), jnp.float32),
            jax.ShapeDtypeStruct((1, 1), jnp.float32),
            jax.ShapeDtypeStruct((1, 1), jnp.float32),
        ],
        scratch_shapes=[pltpu.VMEM((2, E), jnp.float32),
                        pltpu.SMEM((2,), jnp.float32)],
        interpret=_INTERPRET,
    )(flat, w_router)


_PB = 512


def _meta_body(e0_ref, e1_ref, pos_ref, blke_ref, m_ref):
    ef = jnp.concatenate([e0_ref[...], e1_ref[...]], axis=0)
    lanes = jax.lax.broadcasted_iota(jnp.int32, (P, E), 1)
    m_ref[...] = (ef == lanes).astype(jnp.float32)

    counts = jnp.sum(m_ref[...], axis=0)[None, :]
    nb = jnp.floor((counts + (BLK - 1)) * (1.0 / BLK))
    uidx = jax.lax.broadcasted_iota(jnp.int32, (E, E), 0)
    ujdx = jax.lax.broadcasted_iota(jnp.int32, (E, E), 1)
    upper = (uidx < ujdx).astype(jnp.float32)
    offs = BLK * jnp.dot(nb, upper, preferred_element_type=jnp.float32)

    ri = jax.lax.broadcasted_iota(jnp.int32, (_PB, _PB), 0)
    rj = jax.lax.broadcasted_iota(jnp.int32, (_PB, _PB), 1)
    ltri = (rj < ri).astype(jnp.float32)

    def blk_body(b, run):
        mb = m_ref[pl.ds(b * _PB, _PB), :]
        pre = jnp.dot(ltri, mb, preferred_element_type=jnp.float32) + run
        posb = jnp.sum(mb * (pre + offs), axis=1, keepdims=True)
        pos_ref[pl.ds(b * _PB, _PB), :] = posb.astype(jnp.int32)
        return run + jnp.sum(mb, axis=0, keepdims=True)

    lax.fori_loop(0, P // _PB, blk_body, jnp.zeros((1, E), jnp.float32))

    bi = jax.lax.broadcasted_iota(jnp.int32, (NBLOCKS, E), 0).astype(jnp.float32)
    starts = offs * (1.0 / BLK)
    blke = jnp.sum((bi >= starts).astype(jnp.int32), axis=1, keepdims=True) - 1
    blke_ref[...] = blke


def _run_meta(e0, e1):
    return pl.pallas_call(
        _meta_body,
        out_shape=[
            jax.ShapeDtypeStruct((P, 1), jnp.int32),
            jax.ShapeDtypeStruct((NBLOCKS, 1), jnp.int32),
        ],
        scratch_shapes=[pltpu.VMEM((P, E), jnp.float32)],
        interpret=_INTERPRET,
    )(e0, e1)


_SC_MESH = dict(core_axis_name="c", subcore_axis_name="s")
_SPT = NBUF // NW
_GCH = 48
_NCH = _SPT // _GCH


def _sc_dispatch_body(pos_hbm, w_hbm, flat_hbm, xs_hbm, ws_hbm,
                      pos_c, w_c, inv_v, ws_v, rows0, rows1,
                      gsem0, gsem1, wsem0, wsem1):
    wid = lax.axis_index("s") * 2 + lax.axis_index("c")
    base = wid * _SPT

    zf = jnp.zeros((16,), jnp.float32)

    def init_body(i, c):
        spread = (jax.lax.iota(jnp.int32, 16) * 64 + base + i * 16) & (T - 1)
        inv_v[pl.ds(i * 16, 16)] = spread
        ws_v[pl.ds(i * 16, 16)] = zf
        return c

    lax.fori_loop(0, _SPT // 16, init_body, 0)

    def chunk_body(cc, c):
        pltpu.sync_copy(pos_hbm.at[pl.ds(cc * 512, 512)], pos_c)
        pltpu.sync_copy(w_hbm.at[pl.ds(cc * 512, 512)], w_c)

        def sc_body(j, c2):
            idx = pos_c[pl.ds(j * 16, 16)] - base
            wv = w_c[pl.ds(j * 16, 16)]
            msk = (idx >= 0) & (idx < _SPT)
            tok = (jax.lax.iota(jnp.int32, 16) + (cc * 512 + j * 16)) & (T - 1)
            plsc.store_scatter(inv_v, [idx], tok, mask=msk)
            plsc.store_scatter(ws_v, [idx], wv, mask=msk)
            return c2

        lax.fori_loop(0, 512 // 16, sc_body, 0)
        return c

    lax.fori_loop(0, P // 512, chunk_body, 0)

    bufs = (rows0, rows1)
    gsems = (gsem0, gsem1)
    wsems = (wsem0, wsem1)
    gcp = [None, None]
    wcp = [None, None]
    for c in range(_NCH):
        b = c % 2
        if wcp[b] is not None:
            wcp[b].wait()
        gcp[b] = pltpu.async_copy(
            flat_hbm.at[inv_v.at[pl.ds(c * _GCH, _GCH)]], bufs[b], gsems[b])
        if c > 0:
            pb = (c - 1) % 2
            gcp[pb].wait()
            wcp[pb] = pltpu.async_copy(
                bufs[pb], xs_hbm.at[pl.ds(base + (c - 1) * _GCH, _GCH)],
                wsems[pb])
    lb = (_NCH - 1) % 2
    gcp[lb].wait()
    pltpu.sync_copy(bufs[lb], xs_hbm.at[pl.ds(base + (_NCH - 1) * _GCH, _GCH)])
    if wcp[1 - lb] is not None:
        wcp[1 - lb].wait()
    pltpu.sync_copy(ws_v, ws_hbm.at[pl.ds(base, _SPT)])


def _sc_dispatch(posf, wf, flat):
    k = pl.kernel(
        _sc_dispatch_body,
        out_type=[
            jax.ShapeDtypeStruct((NBUF, HIDDEN), jnp.float32),
            jax.ShapeDtypeStruct((NBUF,), jnp.float32),
        ],
        mesh=plsc.VectorSubcoreMesh(**_SC_MESH),
        compiler_params=pltpu.CompilerParams(needs_layout_passes=False),
        scratch_types=[
            pltpu.VMEM((512,), jnp.int32),
            pltpu.VMEM((512,), jnp.float32),
            pltpu.VMEM((_SPT,), jnp.int32),
            pltpu.VMEM((_SPT,), jnp.float32),
            pltpu.VMEM((_GCH, HIDDEN), jnp.float32),
            pltpu.VMEM((_GCH, HIDDEN), jnp.float32),
            pltpu.SemaphoreType.DMA,
            pltpu.SemaphoreType.DMA,
            pltpu.SemaphoreType.DMA,
            pltpu.SemaphoreType.DMA,
        ],
    )
    return k(posf, wf, flat)


_FB = 1024
_FB1 = 2048


def _stage1_body(blke_ref, x_ref, wg_ref, wu_ref, wcol_ref, h_ref):
    x = x_ref[...].astype(jnp.bfloat16)
    g = jnp.dot(x, wg_ref[0].astype(jnp.bfloat16),
                preferred_element_type=jnp.float32)
    u = jnp.dot(x, wu_ref[0].astype(jnp.bfloat16),
                preferred_element_type=jnp.float32)
    h = (g * jax.nn.sigmoid(g)) * u * wcol_ref[...]
    h_ref[...] = h.astype(jnp.bfloat16)


def _run_stage1(blke, xs, wg, wu, wcol):
    grid = (FF // _FB1, NBLOCKS)
    return pl.pallas_call(
        _stage1_body,
        grid_spec=pltpu.PrefetchScalarGridSpec(
            num_scalar_prefetch=1,
            grid=grid,
            in_specs=[
                pl.BlockSpec((BLK, HIDDEN), lambda f, b, s: (b, 0)),
                pl.BlockSpec((1, HIDDEN, _FB1), lambda f, b, s: (s[b], 0, f)),
                pl.BlockSpec((1, HIDDEN, _FB1), lambda f, b, s: (s[b], 0, f)),
                pl.BlockSpec((BLK, 1), lambda f, b, s: (b, 0)),
            ],
            out_specs=pl.BlockSpec((BLK, _FB1), lambda f, b, s: (b, f)),
        ),
        out_shape=jax.ShapeDtypeStruct((NBUF, FF), jnp.bfloat16),
        compiler_params=pltpu.CompilerParams(
            dimension_semantics=("arbitrary", "arbitrary"),
        ),
        interpret=_INTERPRET,
    )(blke, xs, wg, wu, wcol)


def _stage2_body(blke_ref, h_ref, wd_ref, out_ref, acc_ref):
    f = pl.program_id(1)

    @pl.when(f == 0)
    def _():
        acc_ref[...] = jnp.zeros_like(acc_ref)

    acc_ref[...] += jnp.dot(h_ref[...], wd_ref[0].astype(jnp.bfloat16),
                            preferred_element_type=jnp.float32)

    @pl.when(f == pl.num_programs(1) - 1)
    def _():
        out_ref[...] = acc_ref[...]


def _run_stage2(blke, h, wd):
    grid = (NBLOCKS, FF // _FB)
    return pl.pallas_call(
        _stage2_body,
        grid_spec=pltpu.PrefetchScalarGridSpec(
            num_scalar_prefetch=1,
            grid=grid,
            in_specs=[
                pl.BlockSpec((BLK, _FB), lambda b, f, s: (b, f)),
                pl.BlockSpec((1, _FB, HIDDEN), lambda b, f, s: (s[b], f, 0)),
            ],
            out_specs=pl.BlockSpec((BLK, HIDDEN), lambda b, f, s: (b, 0)),
            scratch_shapes=[pltpu.VMEM((BLK, HIDDEN), jnp.float32)],
        ),
        out_shape=jax.ShapeDtypeStruct((NBUF, HIDDEN), jnp.float32),
        compiler_params=pltpu.CompilerParams(
            dimension_semantics=("arbitrary", "arbitrary"),
        ),
        interpret=_INTERPRET,
    )(blke, h, wd)


_TBS = 512


def _shared_body(x_ref, wg_ref, wu_ref, wd_ref, out_ref, acc_ref):
    f = pl.program_id(1)

    @pl.when(f == 0)
    def _():
        acc_ref[...] = jnp.zeros_like(acc_ref)

    x = x_ref[...].astype(jnp.bfloat16)
    g = jnp.dot(x, wg_ref[...].astype(jnp.bfloat16),
                preferred_element_type=jnp.float32)
    u = jnp.dot(x, wu_ref[...].astype(jnp.bfloat16),
                preferred_element_type=jnp.float32)
    h = ((g * jax.nn.sigmoid(g)) * u).astype(jnp.bfloat16)
    acc_ref[...] += jnp.dot(h, wd_ref[...].astype(jnp.bfloat16),
                            preferred_element_type=jnp.float32)

    @pl.when(f == pl.num_programs(1) - 1)
    def _():
        out_ref[...] = 0.5 * acc_ref[...]


def _run_shared(flat, wg_s, wu_s, wd_s):
    grid = (T // _TBS, FF_S // _FB)
    return pl.pallas_call(
        _shared_body,
        grid=grid,
        in_specs=[
            pl.BlockSpec((_TBS, HIDDEN), lambda t, f: (t, 0)),
            pl.BlockSpec((HIDDEN, _FB), lambda t, f: (0, f)),
            pl.BlockSpec((HIDDEN, _FB), lambda t, f: (0, f)),
            pl.BlockSpec((_FB, HIDDEN), lambda t, f: (f, 0)),
        ],
        out_specs=pl.BlockSpec((_TBS, HIDDEN), lambda t, f: (t, 0)),
        out_shape=jax.ShapeDtypeStruct((T, HIDDEN), jnp.float32),
        scratch_shapes=[pltpu.VMEM((_TBS, HIDDEN), jnp.float32)],
        compiler_params=pltpu.CompilerParams(
            dimension_semantics=("parallel", "arbitrary"),
        ),
        interpret=_INTERPRET,
    )(flat, wg_s, wu_s, wd_s)


_TPT = T // NW
_CCH = 32


def _sc_combine_body(outs_hbm, p0_hbm, p1_hbm, y0_hbm, y1_hbm,
                     p0_v, p1_v, r0_v, r1_v, sem0, sem1):
    wid = lax.axis_index("s") * 2 + lax.axis_index("c")
    base = wid * _TPT
    pltpu.sync_copy(p0_hbm.at[pl.ds(base, _TPT)], p0_v)
    pltpu.sync_copy(p1_hbm.at[pl.ds(base, _TPT)], p1_v)

    def c_body(c, carry):
        t0 = base + c * _CCH
        cp0 = pltpu.async_copy(outs_hbm.at[p0_v.at[pl.ds(c * _CCH, _CCH)]],
                               r0_v, sem0)
        cp1 = pltpu.async_copy(outs_hbm.at[p1_v.at[pl.ds(c * _CCH, _CCH)]],
                               r1_v, sem1)
        cp0.wait()
        cp1.wait()
        pltpu.sync_copy(r0_v, y0_hbm.at[pl.ds(t0, _CCH)])
        pltpu.sync_copy(r1_v, y1_hbm.at[pl.ds(t0, _CCH)])
        return carry

    lax.fori_loop(0, _TPT // _CCH, c_body, 0)


def _sc_combine(outs, p0, p1):
    k = pl.kernel(
        _sc_combine_body,
        out_type=[
            jax.ShapeDtypeStruct((T, HIDDEN), jnp.float32),
            jax.ShapeDtypeStruct((T, HIDDEN), jnp.float32),
        ],
        mesh=plsc.VectorSubcoreMesh(**_SC_MESH),
        compiler_params=pltpu.CompilerParams(needs_layout_passes=False),
        scratch_types=[
            pltpu.VMEM((_TPT,), jnp.int32),
            pltpu.VMEM((_TPT,), jnp.int32),
            pltpu.VMEM((_CCH, HIDDEN), jnp.float32),
            pltpu.VMEM((_CCH, HIDDEN), jnp.float32),
            pltpu.SemaphoreType.DMA,
            pltpu.SemaphoreType.DMA,
        ],
    )
    return k(outs, p0, p1)


def _final_body(y0_ref, y1_ref, sh_ref, out_ref):
    out_ref[...] = y0_ref[...] + y1_ref[...] + sh_ref[...]


def _run_final(y0, y1, sh):
    grid = (T // _TBS,)
    spec = pl.BlockSpec((_TBS, HIDDEN), lambda t: (t, 0))
    return pl.pallas_call(
        _final_body,
        grid=grid,
        in_specs=[spec, spec, spec],
        out_specs=spec,
        out_shape=jax.ShapeDtypeStruct((T, HIDDEN), jnp.float32),
        compiler_params=pltpu.CompilerParams(
            dimension_semantics=("parallel",),
        ),
        interpret=_INTERPRET,
    )(y0, y1, sh)


def kernel(hidden_states, W_router, Wg, Wu, Wd, Wg_s, Wu_s, Wd_s):
    Bv, Sv, D = hidden_states.shape
    flat = hidden_states.reshape(-1, D)
    e0, e1, w0, w1, aux, z, ent = _run_router(flat, W_router)
    pos2d, blke2d = _run_meta(e0, e1)
    posf = pos2d.reshape(-1)
    blke = blke2d.reshape(-1)
    wf = jnp.concatenate([w0, w1], axis=0).reshape(-1)
    xs, wslot = _sc_dispatch(posf, wf, flat)
    h = _run_stage1(blke, xs, Wg, Wu, wslot.reshape(NBUF, 1))
    outs = _run_stage2(blke, h, Wd)
    sh = _run_shared(flat, Wg_s, Wu_s, Wd_s)
    y0, y1 = _sc_combine(outs, posf[:T], posf[T:])
    out = _run_final(y0, y1, sh)
    return (out.reshape(Bv, Sv, D), aux[0, 0], z[0, 0], ent[0, 0])
